# Initial kernel scaffold; baseline (speedup 1.0000x reference)
import jax
import jax.numpy as jnp
from jax.experimental import pallas as pl


def kernel(output_stu, output_tch, label):
    raise NotImplementedError("write your pallas kernel here")


# speedup vs baseline: 6.6896x; 6.6896x over previous
import functools
import jax
import jax.numpy as jnp
from jax import lax
from jax.experimental import pallas as pl
from jax.experimental.pallas import tpu as pltpu
from jax.experimental.pallas import tpu_sc as plsc

_ALPHA = 0.5
_T = 4.0
_C = 1000
_CP = 1024
_NEG = -1e30
_B = 4096
_NW = 32
_RPW = _B // _NW

_R = 8


def _tc_kernel(label_ref, stu_ref, tch_ref, out_ref):
    step = pl.program_id(0)
    lane = lax.broadcasted_iota(jnp.int32, (1, _CP), 1)
    lane4 = lax.broadcasted_iota(jnp.int32, (1, 1, 4), 2)
    for r in range(_R):
        lab = label_ref[step * _R + r]
        stu = stu_ref[r, 0, :].reshape(1, _CP)
        tch = tch_ref[r, 0, :].reshape(1, _CP)
        onehot = lane == lab

        m1 = jnp.max(stu)
        z1 = jnp.sum(jnp.exp(stu - m1))
        stu_lab = jnp.sum(jnp.where(onehot, stu, 0.0))
        ce = stu_lab - m1 - jnp.log(z1)

        zT = jnp.sum(jnp.exp((stu - m1) / _T))
        logzT = jnp.log(zT)
        L = (stu - m1) / _T - logzT

        m2 = jnp.max(tch)
        E = jnp.exp((tch - m2) / _T)
        Z = jnp.sum(E)
        logp = (tch - m2) / _T - jnp.log(Z)
        plogp = jnp.sum(E * logp) / Z
        cross0 = jnp.sum(E * L) / Z
        E_lab = jnp.sum(jnp.where(onehot, E, 0.0))
        L_lab = jnp.sum(jnp.where(onehot, L, 0.0))

        partial = plogp - cross0 - (1.0 - E_lab) * L_lab / Z
        m1lz = m1 / _T + logzT

        v = jnp.where(lane4 == 0, ce,
                      jnp.where(lane4 == 1, partial,
                                jnp.where(lane4 == 2, Z, m1lz)))
        out_ref[r] = v[0]


def _sc_kernel(tch_hbm, stu_hbm, lab_hbm, s1_hbm, s2_hbm,
               tchb, stub, labb, Eb, ka, va, kb, vb, kshift, run, s1b, s2b):
    cid = lax.axis_index("c")
    sid = lax.axis_index("s")
    wid = sid * 2 + cid
    base = wid * _RPW
    pltpu.sync_copy(lab_hbm.at[pl.ds(base, _RPW)], labb)

    def row_body(i, _):
        row = base + i
        pltpu.sync_copy(tch_hbm.at[row], tchb)
        pltpu.sync_copy(stu_hbm.at[row], stub)

        def mx(j, m):
            return jnp.maximum(m, tchb[pl.ds(16 * j, 16)])
        m2 = jnp.max(lax.fori_loop(0, 64, mx, jnp.full((16,), _NEG, jnp.float32)))

        def build(j, _):
            t = tchb[pl.ds(16 * j, 16)]
            e = jnp.exp((t - m2) * (1.0 / _T))
            Eb[pl.ds(16 * j, 16)] = e
            ka[pl.ds(16 * j, 16)] = lax.shift_right_logical(
                plsc.bitcast(e, jnp.int32), 16)
            va[pl.ds(16 * j, 16)] = lax.iota(jnp.int32, 16) + 16 * j
            return 0
        lax.fori_loop(0, 64, build, 0)

        for ik, iv, ok, ov, sh, lastp in (
                (ka, va, kb, vb, 0, False),
                (kb, vb, ka, va, 6, False),
                (ka, va, kb, vb, 12, True)):
            for q in range(4):
                run[pl.ds(16 * q, 16)] = jnp.zeros((16,), jnp.int32)

            def hist(j, _, ik=ik, sh=sh):
                d = lax.shift_right_logical(ik[pl.ds(16 * j, 16)], sh) & 63
                cnt, lastm = plsc.scan_count(d)
                plsc.addupdate_scatter(run, [d], cnt, mask=lastm)
                return 0
            lax.fori_loop(0, 64, hist, 0)

            carry = jnp.int32(0)
            for q in range(4):
                h = run[pl.ds(16 * q, 16)]
                inc = plsc.cumsum(h)
                run[pl.ds(16 * q, 16)] = inc - h + carry
                carry = carry + jnp.sum(h)

            def place(j, _, ik=ik, iv=iv, ok=ok, ov=ov, sh=sh, lastp=lastp):
                k = ik[pl.ds(16 * j, 16)]
                v = iv[pl.ds(16 * j, 16)]
                d = lax.shift_right_logical(k, sh) & 63
                cnt, lastm = plsc.scan_count(d)
                dest = plsc.load_gather(run, [d]) + cnt - 1
                plsc.store_scatter(ok, [dest], k)
                plsc.store_scatter(ov, [dest], v)
                if lastp:
                    plsc.store_scatter(kshift, [dest + 1], k)
                plsc.addupdate_scatter(run, [d], cnt, mask=lastm)
                return 0
            lax.fori_loop(0, 64, place, 0)

        labv = plsc.load_gather(labb, [jnp.zeros((16,), jnp.int32) + i])
        labE = plsc.load_gather(Eb, [labv])
        labkey = lax.shift_right_logical(plsc.bitcast(labE, jnp.int32), 16)

        def dsum(j, acc):
            a1, a2 = acc
            k = kb[pl.ds(16 * j, 16)]
            kp = kshift[pl.ds(16 * j, 16)]
            v = vb[pl.ds(16 * j, 16)]
            su = plsc.load_gather(stub, [v])
            e = plsc.bitcast(lax.shift_left(k, 16), jnp.float32)
            ep = plsc.bitcast(lax.shift_left(kp, 16), jnp.float32)
            de = jnp.where(k > labkey, ep - e, 0.0)
            return a1 + de * su, a2 + de
        a1, a2 = lax.fori_loop(
            0, 64, dsum,
            (jnp.zeros((16,), jnp.float32), jnp.zeros((16,), jnp.float32)))

        lane0 = lax.iota(jnp.int32, 16) == 0
        idx16 = jnp.zeros((16,), jnp.int32) + i
        plsc.store_scatter(s1b, [idx16],
                           jnp.zeros((16,), jnp.float32) + jnp.sum(a1),
                           mask=lane0)
        plsc.store_scatter(s2b, [idx16],
                           jnp.zeros((16,), jnp.float32) + jnp.sum(a2),
                           mask=lane0)
        return 0

    lax.fori_loop(0, _RPW, row_body, 0)
    pltpu.sync_copy(s1b, s1_hbm.at[pl.ds(base, _RPW)])
    pltpu.sync_copy(s2b, s2_hbm.at[pl.ds(base, _RPW)])


@jax.jit
def kernel(output_stu, output_tch, label):
    B = output_stu.shape[0]
    pad = _CP - _C
    stu = jnp.pad(output_stu, ((0, 0), (0, pad)), constant_values=_NEG)
    tch = jnp.pad(output_tch, ((0, 0), (0, pad)), constant_values=_NEG)

    mesh = plsc.VectorSubcoreMesh(core_axis_name="c", subcore_axis_name="s")
    sc_call = functools.partial(
        pl.kernel, mesh=mesh,
        compiler_params=pltpu.CompilerParams(needs_layout_passes=False),
        out_type=(jax.ShapeDtypeStruct((B,), jnp.float32),
                  jax.ShapeDtypeStruct((B,), jnp.float32)),
        scratch_types=[
            pltpu.VMEM((_CP,), jnp.float32),
            pltpu.VMEM((_CP,), jnp.float32),
            pltpu.VMEM((_RPW,), jnp.int32),
            pltpu.VMEM((_CP,), jnp.float32),
            pltpu.VMEM((_CP,), jnp.int32),
            pltpu.VMEM((_CP,), jnp.int32),
            pltpu.VMEM((_CP,), jnp.int32),
            pltpu.VMEM((_CP,), jnp.int32),
            pltpu.VMEM((_CP + 16,), jnp.int32),
            pltpu.VMEM((64,), jnp.int32),
            pltpu.VMEM((_RPW,), jnp.float32),
            pltpu.VMEM((_RPW,), jnp.float32),
        ],
    )(_sc_kernel)
    s1, s2 = sc_call(tch, stu, label)

    stu3 = stu.reshape(B, 1, _CP)
    tch3 = tch.reshape(B, 1, _CP)
    grid_spec = pltpu.PrefetchScalarGridSpec(
        num_scalar_prefetch=1,
        grid=(B // _R,),
        in_specs=[
            pl.BlockSpec((_R, 1, _CP), lambda i, lab: (i, 0, 0)),
            pl.BlockSpec((_R, 1, _CP), lambda i, lab: (i, 0, 0)),
        ],
        out_specs=pl.BlockSpec((_R, 1, 4), lambda i, lab: (i, 0, 0)),
    )
    out = pl.pallas_call(
        _tc_kernel,
        grid_spec=grid_spec,
        out_shape=jax.ShapeDtypeStruct((B, 1, 4), jnp.float32),
    )(label, stu3, tch3)

    ce = out[:, 0, 0]
    partial = out[:, 0, 1]
    Z = out[:, 0, 2]
    m1lz = out[:, 0, 3]
    klrow = partial - (s1 / _T - m1lz * s2) / Z

    loss_stu = -jnp.mean(ce)
    loss_tch = jnp.sum(klrow) / (B * _C) * (_T * _T)
    return loss_stu * (1.0 - _ALPHA) + loss_tch * _ALPHA


# speedup vs baseline: 6.6899x; 1.0001x over previous
import functools
import jax
import jax.numpy as jnp
from jax import lax
from jax.experimental import pallas as pl
from jax.experimental.pallas import tpu as pltpu
from jax.experimental.pallas import tpu_sc as plsc

_ALPHA = 0.5
_T = 4.0
_C = 1000
_CP = 1024
_NEG = -1e30
_B = 4096
_NW = 32
_RPW = _B // _NW

_R = 8


def _tc_kernel(label_ref, stu_ref, tch_ref, out_ref):
    step = pl.program_id(0)
    lane = lax.broadcasted_iota(jnp.int32, (1, _CP), 1)
    lane4 = lax.broadcasted_iota(jnp.int32, (1, 1, 4), 2)
    for r in range(_R):
        lab = label_ref[step * _R + r]
        stu = stu_ref[r, 0, :].reshape(1, _CP)
        tch = tch_ref[r, 0, :].reshape(1, _CP)
        onehot = lane == lab

        m1 = jnp.max(stu)
        z1 = jnp.sum(jnp.exp(stu - m1))
        stu_lab = jnp.sum(jnp.where(onehot, stu, 0.0))
        ce = stu_lab - m1 - jnp.log(z1)

        zT = jnp.sum(jnp.exp((stu - m1) / _T))
        logzT = jnp.log(zT)
        L = (stu - m1) / _T - logzT

        m2 = jnp.max(tch)
        E = jnp.exp((tch - m2) / _T)
        Z = jnp.sum(E)
        logp = (tch - m2) / _T - jnp.log(Z)
        plogp = jnp.sum(E * logp) / Z
        cross0 = jnp.sum(E * L) / Z
        E_lab = jnp.sum(jnp.where(onehot, E, 0.0))
        L_lab = jnp.sum(jnp.where(onehot, L, 0.0))

        partial = plogp - cross0 - (1.0 - E_lab) * L_lab / Z
        m1lz = m1 / _T + logzT

        v = jnp.where(lane4 == 0, ce,
                      jnp.where(lane4 == 1, partial,
                                jnp.where(lane4 == 2, Z, m1lz)))
        out_ref[r] = v[0]


def _sc_kernel(tch_hbm, stu_hbm, lab_hbm, s12_hbm,
               tchb, stub, labb, Eb, ka, va, kb, vb, kshift, run, s1b, s2b):
    cid = lax.axis_index("c")
    sid = lax.axis_index("s")
    wid = sid * 2 + cid
    base = wid * _RPW
    pltpu.sync_copy(lab_hbm.at[pl.ds(base, _RPW)], labb)

    def row_body(i, _):
        row = base + i
        pltpu.sync_copy(tch_hbm.at[row], tchb)
        pltpu.sync_copy(stu_hbm.at[row], stub)

        def mx(j, m):
            return jnp.maximum(m, tchb[pl.ds(16 * j, 16)])
        m2 = jnp.max(lax.fori_loop(0, 64, mx, jnp.full((16,), _NEG, jnp.float32)))

        def build(j, _):
            t = tchb[pl.ds(16 * j, 16)]
            e = jnp.exp((t - m2) * (1.0 / _T))
            Eb[pl.ds(16 * j, 16)] = e
            ka[pl.ds(16 * j, 16)] = lax.shift_right_logical(
                plsc.bitcast(e, jnp.int32), 16)
            va[pl.ds(16 * j, 16)] = lax.iota(jnp.int32, 16) + 16 * j
            return 0
        lax.fori_loop(0, 64, build, 0)

        for ik, iv, ok, ov, sh, lastp in (
                (ka, va, kb, vb, 0, False),
                (kb, vb, ka, va, 8, True)):
            for q in range(16):
                run[pl.ds(16 * q, 16)] = jnp.zeros((16,), jnp.int32)

            def hist(j, _, ik=ik, sh=sh):
                d = lax.shift_right_logical(ik[pl.ds(16 * j, 16)], sh) & 255
                cnt, lastm = plsc.scan_count(d)
                plsc.addupdate_scatter(run, [d], cnt, mask=lastm)
                return 0
            lax.fori_loop(0, 64, hist, 0)

            carry = jnp.int32(0)
            for q in range(16):
                h = run[pl.ds(16 * q, 16)]
                inc = plsc.cumsum(h)
                run[pl.ds(16 * q, 16)] = inc - h + carry
                carry = carry + jnp.sum(h)

            def place(j, _, ik=ik, iv=iv, ok=ok, ov=ov, sh=sh, lastp=lastp):
                k = ik[pl.ds(16 * j, 16)]
                v = iv[pl.ds(16 * j, 16)]
                d = lax.shift_right_logical(k, sh) & 255
                cnt, lastm = plsc.scan_count(d)
                dest = plsc.load_gather(run, [d]) + cnt - 1
                plsc.store_scatter(ok, [dest], k)
                plsc.store_scatter(ov, [dest], v)
                if lastp:
                    plsc.store_scatter(kshift, [dest + 1], k)
                plsc.addupdate_scatter(run, [d], cnt, mask=lastm)
                return 0
            lax.fori_loop(0, 64, place, 0)

        labv = plsc.load_gather(labb, [jnp.zeros((16,), jnp.int32) + i])
        labE = plsc.load_gather(Eb, [labv])
        labkey = lax.shift_right_logical(plsc.bitcast(labE, jnp.int32), 16)

        def dsum(j, acc):
            a1, a2 = acc
            k = ka[pl.ds(16 * j, 16)]
            kp = kshift[pl.ds(16 * j, 16)]
            v = va[pl.ds(16 * j, 16)]
            su = plsc.load_gather(stub, [v])
            e = plsc.bitcast(lax.shift_left(k, 16), jnp.float32)
            ep = plsc.bitcast(lax.shift_left(kp, 16), jnp.float32)
            de = jnp.where(k > labkey, ep - e, 0.0)
            return a1 + de * su, a2 + de
        a1, a2 = lax.fori_loop(
            0, 64, dsum,
            (jnp.zeros((16,), jnp.float32), jnp.zeros((16,), jnp.float32)))

        lane0 = lax.iota(jnp.int32, 16) == 0
        idx16 = jnp.zeros((16,), jnp.int32) + i
        plsc.store_scatter(s1b, [idx16],
                           jnp.zeros((16,), jnp.float32) + jnp.sum(a1),
                           mask=lane0)
        plsc.store_scatter(s2b, [idx16],
                           jnp.zeros((16,), jnp.float32) + jnp.sum(a2),
                           mask=lane0)
        return 0

    lax.fori_loop(0, _RPW, row_body, 0)
    pltpu.sync_copy(s1b, s12_hbm.at[0, pl.ds(base, _RPW)])
    pltpu.sync_copy(s2b, s12_hbm.at[1, pl.ds(base, _RPW)])


@jax.jit
def kernel(output_stu, output_tch, label):
    B = output_stu.shape[0]
    pad = _CP - _C
    stu = jnp.pad(output_stu, ((0, 0), (0, pad)), constant_values=_NEG)
    tch = jnp.pad(output_tch, ((0, 0), (0, pad)), constant_values=_NEG)

    mesh = plsc.VectorSubcoreMesh(core_axis_name="c", subcore_axis_name="s")
    sc_call = functools.partial(
        pl.kernel, mesh=mesh,
        compiler_params=pltpu.CompilerParams(needs_layout_passes=False),
        out_type=jax.ShapeDtypeStruct((2, B), jnp.float32),
        scratch_types=[
            pltpu.VMEM((_CP,), jnp.float32),
            pltpu.VMEM((_CP,), jnp.float32),
            pltpu.VMEM((_RPW,), jnp.int32),
            pltpu.VMEM((_CP,), jnp.float32),
            pltpu.VMEM((_CP,), jnp.int32),
            pltpu.VMEM((_CP,), jnp.int32),
            pltpu.VMEM((_CP,), jnp.int32),
            pltpu.VMEM((_CP,), jnp.int32),
            pltpu.VMEM((_CP + 16,), jnp.int32),
            pltpu.VMEM((256,), jnp.int32),
            pltpu.VMEM((_RPW,), jnp.float32),
            pltpu.VMEM((_RPW,), jnp.float32),
        ],
    )(_sc_kernel)
    s12 = sc_call(tch, stu, label)
    s1 = s12[0]
    s2 = s12[1]

    stu3 = stu.reshape(B, 1, _CP)
    tch3 = tch.reshape(B, 1, _CP)
    grid_spec = pltpu.PrefetchScalarGridSpec(
        num_scalar_prefetch=1,
        grid=(B // _R,),
        in_specs=[
            pl.BlockSpec((_R, 1, _CP), lambda i, lab: (i, 0, 0)),
            pl.BlockSpec((_R, 1, _CP), lambda i, lab: (i, 0, 0)),
        ],
        out_specs=pl.BlockSpec((_R, 1, 4), lambda i, lab: (i, 0, 0)),
    )
    out = pl.pallas_call(
        _tc_kernel,
        grid_spec=grid_spec,
        out_shape=jax.ShapeDtypeStruct((B, 1, 4), jnp.float32),
    )(label, stu3, tch3)

    ce = out[:, 0, 0]
    partial = out[:, 0, 1]
    Z = out[:, 0, 2]
    m1lz = out[:, 0, 3]
    klrow = partial - (s1 / _T - m1lz * s2) / Z

    loss_stu = -jnp.mean(ce)
    loss_tch = jnp.sum(klrow) / (B * _C) * (_T * _T)
    return loss_stu * (1.0 - _ALPHA) + loss_tch * _ALPHA


# speedup vs baseline: 7.0655x; 1.0562x over previous
import functools
import jax
import jax.numpy as jnp
from jax import lax
from jax.experimental import pallas as pl
from jax.experimental.pallas import tpu as pltpu
from jax.experimental.pallas import tpu_sc as plsc

_ALPHA = 0.5
_T = 4.0
_C = 1000
_CP = 1024
_NEG = -1e30
_B = 4096
_NW = 32
_RPW = _B // _NW

_R = 8


def _tc_kernel(label_ref, stu_ref, tch_ref, out_ref):
    step = pl.program_id(0)
    isub = lax.broadcasted_iota(jnp.int32, (8, 128), 0)
    ilane = lax.broadcasted_iota(jnp.int32, (8, 128), 1)
    lane4 = lax.broadcasted_iota(jnp.int32, (1, 1, 4), 2)
    for r in range(_R):
        lab = label_ref[step * _R + r]
        stu = stu_ref[r]
        tch = tch_ref[r]
        onehot = (isub == lab // 128) & (ilane == lab % 128)

        m1 = jnp.max(stu)
        z1 = jnp.sum(jnp.exp(stu - m1))
        stu_lab = jnp.sum(jnp.where(onehot, stu, 0.0))
        ce = stu_lab - m1 - jnp.log(z1)

        zT = jnp.sum(jnp.exp((stu - m1) / _T))
        logzT = jnp.log(zT)
        L = (stu - m1) / _T - logzT

        m2 = jnp.max(tch)
        E = jnp.exp((tch - m2) / _T)
        Z = jnp.sum(E)
        logp = (tch - m2) / _T - jnp.log(Z)
        plogp = jnp.sum(E * logp) / Z
        cross0 = jnp.sum(E * L) / Z
        E_lab = jnp.sum(jnp.where(onehot, E, 0.0))
        L_lab = jnp.sum(jnp.where(onehot, L, 0.0))

        partial = plogp - cross0 - (1.0 - E_lab) * L_lab / Z
        m1lz = m1 / _T + logzT

        v = jnp.where(lane4 == 0, ce,
                      jnp.where(lane4 == 1, partial,
                                jnp.where(lane4 == 2, Z, m1lz)))
        out_ref[r] = v[0]


def _sc_kernel(tch_hbm, stu_hbm, lab_hbm, s12_hbm,
               tchb, stub, labb, Eb, ka, va, kb, vb, kshift, run, s1b, s2b):
    cid = lax.axis_index("c")
    sid = lax.axis_index("s")
    wid = sid * 2 + cid
    base = wid * _RPW
    pltpu.sync_copy(lab_hbm.at[pl.ds(base, _RPW)], labb)

    def row_body(i, _):
        row = base + i
        pltpu.sync_copy(tch_hbm.at[row], tchb)
        pltpu.sync_copy(stu_hbm.at[row], stub)

        def mx(j, m):
            return jnp.maximum(m, tchb[pl.ds(16 * j, 16)])
        m2 = jnp.max(lax.fori_loop(0, 64, mx, jnp.full((16,), _NEG, jnp.float32)))

        def build(j, _):
            t = tchb[pl.ds(16 * j, 16)]
            e = jnp.exp((t - m2) * (1.0 / _T))
            Eb[pl.ds(16 * j, 16)] = e
            ka[pl.ds(16 * j, 16)] = lax.shift_right_logical(
                plsc.bitcast(e, jnp.int32), 16)
            va[pl.ds(16 * j, 16)] = lax.iota(jnp.int32, 16) + 16 * j
            return 0
        lax.fori_loop(0, 64, build, 0)

        for ik, iv, ok, ov, sh, lastp in (
                (ka, va, kb, vb, 0, False),
                (kb, vb, ka, va, 8, True)):
            for q in range(16):
                run[pl.ds(16 * q, 16)] = jnp.zeros((16,), jnp.int32)

            def hist(j, _, ik=ik, sh=sh):
                d = lax.shift_right_logical(ik[pl.ds(16 * j, 16)], sh) & 255
                cnt, lastm = plsc.scan_count(d)
                plsc.addupdate_scatter(run, [d], cnt, mask=lastm)
                return 0
            lax.fori_loop(0, 64, hist, 0)

            carry = jnp.int32(0)
            for q in range(16):
                h = run[pl.ds(16 * q, 16)]
                inc = plsc.cumsum(h)
                run[pl.ds(16 * q, 16)] = inc - h + carry
                carry = carry + jnp.sum(h)

            def place(j, _, ik=ik, iv=iv, ok=ok, ov=ov, sh=sh, lastp=lastp):
                k = ik[pl.ds(16 * j, 16)]
                v = iv[pl.ds(16 * j, 16)]
                d = lax.shift_right_logical(k, sh) & 255
                cnt, lastm = plsc.scan_count(d)
                dest = plsc.load_gather(run, [d]) + cnt - 1
                plsc.store_scatter(ok, [dest], k)
                plsc.store_scatter(ov, [dest], v)
                if lastp:
                    plsc.store_scatter(kshift, [dest + 1], k)
                plsc.addupdate_scatter(run, [d], cnt, mask=lastm)
                return 0
            lax.fori_loop(0, 64, place, 0)

        labv = plsc.load_gather(labb, [jnp.zeros((16,), jnp.int32) + i])
        labE = plsc.load_gather(Eb, [labv])
        labkey = lax.shift_right_logical(plsc.bitcast(labE, jnp.int32), 16)

        def dsum(j, acc):
            a1, a2 = acc
            k = ka[pl.ds(16 * j, 16)]
            kp = kshift[pl.ds(16 * j, 16)]
            v = va[pl.ds(16 * j, 16)]
            su = plsc.load_gather(stub, [v])
            e = plsc.bitcast(lax.shift_left(k, 16), jnp.float32)
            ep = plsc.bitcast(lax.shift_left(kp, 16), jnp.float32)
            de = jnp.where(k > labkey, ep - e, 0.0)
            return a1 + de * su, a2 + de
        a1, a2 = lax.fori_loop(
            0, 64, dsum,
            (jnp.zeros((16,), jnp.float32), jnp.zeros((16,), jnp.float32)))

        lane0 = lax.iota(jnp.int32, 16) == 0
        idx16 = jnp.zeros((16,), jnp.int32) + i
        plsc.store_scatter(s1b, [idx16],
                           jnp.zeros((16,), jnp.float32) + jnp.sum(a1),
                           mask=lane0)
        plsc.store_scatter(s2b, [idx16],
                           jnp.zeros((16,), jnp.float32) + jnp.sum(a2),
                           mask=lane0)
        return 0

    lax.fori_loop(0, _RPW, row_body, 0)
    pltpu.sync_copy(s1b, s12_hbm.at[0, pl.ds(base, _RPW)])
    pltpu.sync_copy(s2b, s12_hbm.at[1, pl.ds(base, _RPW)])


@jax.jit
def kernel(output_stu, output_tch, label):
    B = output_stu.shape[0]
    pad = _CP - _C
    stu = jnp.pad(output_stu, ((0, 0), (0, pad)), constant_values=_NEG)
    tch = jnp.pad(output_tch, ((0, 0), (0, pad)), constant_values=_NEG)

    mesh = plsc.VectorSubcoreMesh(core_axis_name="c", subcore_axis_name="s")
    sc_call = functools.partial(
        pl.kernel, mesh=mesh,
        compiler_params=pltpu.CompilerParams(needs_layout_passes=False),
        out_type=jax.ShapeDtypeStruct((2, B), jnp.float32),
        scratch_types=[
            pltpu.VMEM((_CP,), jnp.float32),
            pltpu.VMEM((_CP,), jnp.float32),
            pltpu.VMEM((_RPW,), jnp.int32),
            pltpu.VMEM((_CP,), jnp.float32),
            pltpu.VMEM((_CP,), jnp.int32),
            pltpu.VMEM((_CP,), jnp.int32),
            pltpu.VMEM((_CP,), jnp.int32),
            pltpu.VMEM((_CP,), jnp.int32),
            pltpu.VMEM((_CP + 16,), jnp.int32),
            pltpu.VMEM((256,), jnp.int32),
            pltpu.VMEM((_RPW,), jnp.float32),
            pltpu.VMEM((_RPW,), jnp.float32),
        ],
    )(_sc_kernel)
    s12 = sc_call(tch, stu, label)
    s1 = s12[0]
    s2 = s12[1]

    stu3 = stu.reshape(B, 8, 128)
    tch3 = tch.reshape(B, 8, 128)
    grid_spec = pltpu.PrefetchScalarGridSpec(
        num_scalar_prefetch=1,
        grid=(B // _R,),
        in_specs=[
            pl.BlockSpec((_R, 8, 128), lambda i, lab: (i, 0, 0)),
            pl.BlockSpec((_R, 8, 128), lambda i, lab: (i, 0, 0)),
        ],
        out_specs=pl.BlockSpec((_R, 1, 4), lambda i, lab: (i, 0, 0)),
    )
    out = pl.pallas_call(
        _tc_kernel,
        grid_spec=grid_spec,
        out_shape=jax.ShapeDtypeStruct((B, 1, 4), jnp.float32),
    )(label, stu3, tch3)

    ce = out[:, 0, 0]
    partial = out[:, 0, 1]
    Z = out[:, 0, 2]
    m1lz = out[:, 0, 3]
    klrow = partial - (s1 / _T - m1lz * s2) / Z

    loss_stu = -jnp.mean(ce)
    loss_tch = jnp.sum(klrow) / (B * _C) * (_T * _T)
    return loss_stu * (1.0 - _ALPHA) + loss_tch * _ALPHA


# speedup vs baseline: 16.0486x; 2.2714x over previous
import functools
import jax
import jax.numpy as jnp
from jax import lax
from jax.experimental import pallas as pl
from jax.experimental.pallas import tpu as pltpu
from jax.experimental.pallas import tpu_sc as plsc

_ALPHA = 0.5
_T = 4.0
_C = 1000
_CP = 1024
_NEG = -1e30
_B = 4096
_NW = 32
_RPW = _B // _NW

_R = 8


def _rsum(x):
    return jnp.sum(jnp.sum(x, axis=2, keepdims=True), axis=1, keepdims=True)


def _rmax(x):
    return jnp.max(jnp.max(x, axis=2, keepdims=True), axis=1, keepdims=True)


def _tc_kernel(label_ref, stu_ref, tch_ref, out_ref):
    step = pl.program_id(0)
    isub = lax.broadcasted_iota(jnp.int32, (_R, 8, 128), 1)
    ilane = lax.broadcasted_iota(jnp.int32, (_R, 8, 128), 2)
    iowr = lax.broadcasted_iota(jnp.int32, (_R, 1, 1), 0)
    lane4 = lax.broadcasted_iota(jnp.int32, (1, 1, 4), 2)

    labv = jnp.zeros((_R, 1, 1), jnp.int32)
    for r in range(_R):
        labv = jnp.where(iowr == r, label_ref[step * _R + r], labv)

    stu = stu_ref[...]
    tch = tch_ref[...]
    onehot = (isub == labv // 128) & (ilane == labv % 128)

    m1 = _rmax(stu)
    z1 = _rsum(jnp.exp(stu - m1))
    stu_lab = _rsum(jnp.where(onehot, stu, 0.0))
    ce = stu_lab - m1 - jnp.log(z1)

    zT = _rsum(jnp.exp((stu - m1) / _T))
    logzT = jnp.log(zT)
    L = (stu - m1) / _T - logzT

    m2 = _rmax(tch)
    E = jnp.exp((tch - m2) / _T)
    Z = _rsum(E)
    logp = (tch - m2) / _T - jnp.log(Z)
    plogp = _rsum(E * logp) / Z
    cross0 = _rsum(E * L) / Z
    E_lab = _rsum(jnp.where(onehot, E, 0.0))
    L_lab = _rsum(jnp.where(onehot, L, 0.0))

    partial = plogp - cross0 - (1.0 - E_lab) * L_lab / Z
    m1lz = m1 / _T + logzT

    out_ref[...] = jnp.where(lane4 == 0, ce,
                             jnp.where(lane4 == 1, partial,
                                       jnp.where(lane4 == 2, Z, m1lz)))


def _sc_kernel(tch_hbm, stu_hbm, lab_hbm, s12_hbm,
               tchb, stub, labb, Eb, ka, va, kb, vb, kshift, run, s1b, s2b):
    cid = lax.axis_index("c")
    sid = lax.axis_index("s")
    wid = sid * 2 + cid
    base = wid * _RPW
    pltpu.sync_copy(lab_hbm.at[pl.ds(base, _RPW)], labb)

    def row_body(i, _):
        row = base + i
        pltpu.sync_copy(tch_hbm.at[row], tchb)
        pltpu.sync_copy(stu_hbm.at[row], stub)

        def mx(j, m):
            return jnp.maximum(m, tchb[pl.ds(16 * j, 16)])
        m2 = jnp.max(lax.fori_loop(0, 64, mx, jnp.full((16,), _NEG, jnp.float32)))

        def build(j, _):
            t = tchb[pl.ds(16 * j, 16)]
            e = jnp.exp((t - m2) * (1.0 / _T))
            Eb[pl.ds(16 * j, 16)] = e
            ka[pl.ds(16 * j, 16)] = lax.shift_right_logical(
                plsc.bitcast(e, jnp.int32), 16)
            va[pl.ds(16 * j, 16)] = lax.iota(jnp.int32, 16) + 16 * j
            return 0
        lax.fori_loop(0, 64, build, 0)

        for ik, iv, ok, ov, sh, lastp in (
                (ka, va, kb, vb, 0, False),
                (kb, vb, ka, va, 8, True)):
            for q in range(16):
                run[pl.ds(16 * q, 16)] = jnp.zeros((16,), jnp.int32)

            def hist(j, _, ik=ik, sh=sh):
                d = lax.shift_right_logical(ik[pl.ds(16 * j, 16)], sh) & 255
                cnt, lastm = plsc.scan_count(d)
                plsc.addupdate_scatter(run, [d], cnt, mask=lastm)
                return 0
            lax.fori_loop(0, 64, hist, 0)

            carry = jnp.int32(0)
            for q in range(16):
                h = run[pl.ds(16 * q, 16)]
                inc = plsc.cumsum(h)
                run[pl.ds(16 * q, 16)] = inc - h + carry
                carry = carry + jnp.sum(h)

            def place(j, _, ik=ik, iv=iv, ok=ok, ov=ov, sh=sh, lastp=lastp):
                k = ik[pl.ds(16 * j, 16)]
                v = iv[pl.ds(16 * j, 16)]
                d = lax.shift_right_logical(k, sh) & 255
                cnt, lastm = plsc.scan_count(d)
                dest = plsc.load_gather(run, [d]) + cnt - 1
                plsc.store_scatter(ok, [dest], k)
                plsc.store_scatter(ov, [dest], v)
                if lastp:
                    plsc.store_scatter(kshift, [dest + 1], k)
                plsc.addupdate_scatter(run, [d], cnt, mask=lastm)
                return 0
            lax.fori_loop(0, 64, place, 0)

        labv = plsc.load_gather(labb, [jnp.zeros((16,), jnp.int32) + i])
        labE = plsc.load_gather(Eb, [labv])
        labkey = lax.shift_right_logical(plsc.bitcast(labE, jnp.int32), 16)

        def dsum(j, acc):
            a1, a2 = acc
            k = ka[pl.ds(16 * j, 16)]
            kp = kshift[pl.ds(16 * j, 16)]
            v = va[pl.ds(16 * j, 16)]
            su = plsc.load_gather(stub, [v])
            e = plsc.bitcast(lax.shift_left(k, 16), jnp.float32)
            ep = plsc.bitcast(lax.shift_left(kp, 16), jnp.float32)
            de = jnp.where(k > labkey, ep - e, 0.0)
            return a1 + de * su, a2 + de
        a1, a2 = lax.fori_loop(
            0, 64, dsum,
            (jnp.zeros((16,), jnp.float32), jnp.zeros((16,), jnp.float32)))

        lane0 = lax.iota(jnp.int32, 16) == 0
        idx16 = jnp.zeros((16,), jnp.int32) + i
        plsc.store_scatter(s1b, [idx16],
                           jnp.zeros((16,), jnp.float32) + jnp.sum(a1),
                           mask=lane0)
        plsc.store_scatter(s2b, [idx16],
                           jnp.zeros((16,), jnp.float32) + jnp.sum(a2),
                           mask=lane0)
        return 0

    lax.fori_loop(0, _RPW, row_body, 0)
    pltpu.sync_copy(s1b, s12_hbm.at[0, pl.ds(base, _RPW)])
    pltpu.sync_copy(s2b, s12_hbm.at[1, pl.ds(base, _RPW)])


@jax.jit
def kernel(output_stu, output_tch, label):
    B = output_stu.shape[0]
    pad = _CP - _C
    stu = jnp.pad(output_stu, ((0, 0), (0, pad)), constant_values=_NEG)
    tch = jnp.pad(output_tch, ((0, 0), (0, pad)), constant_values=_NEG)

    mesh = plsc.VectorSubcoreMesh(core_axis_name="c", subcore_axis_name="s")
    sc_call = functools.partial(
        pl.kernel, mesh=mesh,
        compiler_params=pltpu.CompilerParams(needs_layout_passes=False),
        out_type=jax.ShapeDtypeStruct((2, B), jnp.float32),
        scratch_types=[
            pltpu.VMEM((_CP,), jnp.float32),
            pltpu.VMEM((_CP,), jnp.float32),
            pltpu.VMEM((_RPW,), jnp.int32),
            pltpu.VMEM((_CP,), jnp.float32),
            pltpu.VMEM((_CP,), jnp.int32),
            pltpu.VMEM((_CP,), jnp.int32),
            pltpu.VMEM((_CP,), jnp.int32),
            pltpu.VMEM((_CP,), jnp.int32),
            pltpu.VMEM((_CP + 16,), jnp.int32),
            pltpu.VMEM((256,), jnp.int32),
            pltpu.VMEM((_RPW,), jnp.float32),
            pltpu.VMEM((_RPW,), jnp.float32),
        ],
    )(_sc_kernel)
    s12 = sc_call(tch, stu, label)
    s1 = s12[0]
    s2 = s12[1]

    stu3 = stu.reshape(B, 8, 128)
    tch3 = tch.reshape(B, 8, 128)
    grid_spec = pltpu.PrefetchScalarGridSpec(
        num_scalar_prefetch=1,
        grid=(B // _R,),
        in_specs=[
            pl.BlockSpec((_R, 8, 128), lambda i, lab: (i, 0, 0)),
            pl.BlockSpec((_R, 8, 128), lambda i, lab: (i, 0, 0)),
        ],
        out_specs=pl.BlockSpec((_R, 1, 4), lambda i, lab: (i, 0, 0)),
    )
    out = pl.pallas_call(
        _tc_kernel,
        grid_spec=grid_spec,
        out_shape=jax.ShapeDtypeStruct((B, 1, 4), jnp.float32),
    )(label, stu3, tch3)

    ce = out[:, 0, 0]
    partial = out[:, 0, 1]
    Z = out[:, 0, 2]
    m1lz = out[:, 0, 3]
    klrow = partial - (s1 / _T - m1lz * s2) / Z

    loss_stu = -jnp.mean(ce)
    loss_tch = jnp.sum(klrow) / (B * _C) * (_T * _T)
    return loss_stu * (1.0 - _ALPHA) + loss_tch * _ALPHA


# speedup vs baseline: 16.9420x; 1.0557x over previous
import functools
import jax
import jax.numpy as jnp
from jax import lax
from jax.experimental import pallas as pl
from jax.experimental.pallas import tpu as pltpu
from jax.experimental.pallas import tpu_sc as plsc

_ALPHA = 0.5
_T = 4.0
_C = 1000
_CP = 1024
_NEG = -1e30
_B = 4096
_NW = 32
_RPW = _B // _NW

_R = 8


def _rsum(x):
    return jnp.sum(jnp.sum(x, axis=2, keepdims=True), axis=1, keepdims=True)


def _rmax(x):
    return jnp.max(jnp.max(x, axis=2, keepdims=True), axis=1, keepdims=True)


def _tc_kernel(label_ref, stu_ref, tch_ref, out_ref):
    step = pl.program_id(0)
    isub = lax.broadcasted_iota(jnp.int32, (_R, 8, 128), 1)
    ilane = lax.broadcasted_iota(jnp.int32, (_R, 8, 128), 2)
    iowr = lax.broadcasted_iota(jnp.int32, (_R, 1, 1), 0)
    lane4 = lax.broadcasted_iota(jnp.int32, (1, 1, 4), 2)

    labv = jnp.zeros((_R, 1, 1), jnp.int32)
    for r in range(_R):
        labv = jnp.where(iowr == r, label_ref[step * _R + r], labv)

    stu = stu_ref[...]
    tch = tch_ref[...]
    onehot = (isub == labv // 128) & (ilane == labv % 128)

    m1 = _rmax(stu)
    z1 = _rsum(jnp.exp(stu - m1))
    stu_lab = _rsum(jnp.where(onehot, stu, 0.0))
    ce = stu_lab - m1 - jnp.log(z1)

    zT = _rsum(jnp.exp((stu - m1) / _T))
    logzT = jnp.log(zT)
    L = (stu - m1) / _T - logzT

    m2 = _rmax(tch)
    E = jnp.exp((tch - m2) / _T)
    Z = _rsum(E)
    logp = (tch - m2) / _T - jnp.log(Z)
    plogp = _rsum(E * logp) / Z
    cross0 = _rsum(E * L) / Z
    E_lab = _rsum(jnp.where(onehot, E, 0.0))
    L_lab = _rsum(jnp.where(onehot, L, 0.0))

    partial = plogp - cross0 - (1.0 - E_lab) * L_lab / Z
    m1lz = m1 / _T + logzT

    out_ref[...] = jnp.where(lane4 == 0, ce,
                             jnp.where(lane4 == 1, partial,
                                       jnp.where(lane4 == 2, Z, m1lz)))


def _sc_kernel(tch_hbm, stu_hbm, lab_hbm, s12_hbm,
               tchb, stub, labb, Eb, ka, va, kb, vb, kshift, run, s1b, s2b):
    cid = lax.axis_index("c")
    sid = lax.axis_index("s")
    wid = sid * 2 + cid
    base = wid * _RPW
    pltpu.sync_copy(lab_hbm.at[pl.ds(base, _RPW)], labb)

    def row_body(i, _):
        row = base + i
        pltpu.sync_copy(tch_hbm.at[row], tchb)
        pltpu.sync_copy(stu_hbm.at[row], stub)

        def mx(j, m):
            m0 = jnp.maximum(m[0], tchb[pl.ds(64 * j, 16)])
            m1_ = jnp.maximum(m[1], tchb[pl.ds(64 * j + 16, 16)])
            m2_ = jnp.maximum(m[2], tchb[pl.ds(64 * j + 32, 16)])
            m3 = jnp.maximum(m[3], tchb[pl.ds(64 * j + 48, 16)])
            return (m0, m1_, m2_, m3)
        mi = jnp.full((16,), _NEG, jnp.float32)
        mt = lax.fori_loop(0, 16, mx, (mi, mi, mi, mi))
        m2 = jnp.max(jnp.maximum(jnp.maximum(mt[0], mt[1]),
                                 jnp.maximum(mt[2], mt[3])))

        def build(j, _):
            for u in range(4):
                o = 64 * j + 16 * u
                t = tchb[pl.ds(o, 16)]
                e = jnp.exp((t - m2) * (1.0 / _T))
                Eb[pl.ds(o, 16)] = e
                ka[pl.ds(o, 16)] = lax.shift_right_logical(
                    plsc.bitcast(e, jnp.int32), 16)
                va[pl.ds(o, 16)] = lax.iota(jnp.int32, 16) + o
            return 0
        lax.fori_loop(0, 16, build, 0)

        for ik, iv, ok, ov, sh, lastp in (
                (ka, va, kb, vb, 0, False),
                (kb, vb, ka, va, 8, True)):
            for q in range(16):
                run[pl.ds(16 * q, 16)] = jnp.zeros((16,), jnp.int32)

            def hist(j, _, ik=ik, sh=sh):
                for u in range(4):
                    o = 64 * j + 16 * u
                    d = lax.shift_right_logical(ik[pl.ds(o, 16)], sh) & 255
                    cnt, lastm = plsc.scan_count(d)
                    plsc.addupdate_scatter(run, [d], cnt, mask=lastm)
                return 0
            lax.fori_loop(0, 16, hist, 0)

            carry = jnp.int32(0)
            for q in range(16):
                h = run[pl.ds(16 * q, 16)]
                inc = plsc.cumsum(h)
                run[pl.ds(16 * q, 16)] = inc - h + carry
                carry = carry + jnp.sum(h)

            def place(j, _, ik=ik, iv=iv, ok=ok, ov=ov, sh=sh, lastp=lastp):
                for u in range(2):
                    o = 32 * j + 16 * u
                    k = ik[pl.ds(o, 16)]
                    v = iv[pl.ds(o, 16)]
                    d = lax.shift_right_logical(k, sh) & 255
                    cnt, lastm = plsc.scan_count(d)
                    dest = plsc.load_gather(run, [d]) + cnt - 1
                    plsc.store_scatter(ok, [dest], k)
                    plsc.store_scatter(ov, [dest], v)
                    if lastp:
                        plsc.store_scatter(kshift, [dest + 1], k)
                    plsc.addupdate_scatter(run, [d], cnt, mask=lastm)
                return 0
            lax.fori_loop(0, 32, place, 0)

        labv = plsc.load_gather(labb, [jnp.zeros((16,), jnp.int32) + i])
        labE = plsc.load_gather(Eb, [labv])
        labkey = lax.shift_right_logical(plsc.bitcast(labE, jnp.int32), 16)

        def dsum(j, acc):
            a1, a2 = acc
            for u in range(4):
                o = 64 * j + 16 * u
                k = ka[pl.ds(o, 16)]
                kp = kshift[pl.ds(o, 16)]
                v = va[pl.ds(o, 16)]
                su = plsc.load_gather(stub, [v])
                e = plsc.bitcast(lax.shift_left(k, 16), jnp.float32)
                ep = plsc.bitcast(lax.shift_left(kp, 16), jnp.float32)
                de = jnp.where(k > labkey, ep - e, 0.0)
                a1 = a1 + de * su
                a2 = a2 + de
            return a1, a2
        a1, a2 = lax.fori_loop(
            0, 16, dsum,
            (jnp.zeros((16,), jnp.float32), jnp.zeros((16,), jnp.float32)))

        lane0 = lax.iota(jnp.int32, 16) == 0
        idx16 = jnp.zeros((16,), jnp.int32) + i
        plsc.store_scatter(s1b, [idx16],
                           jnp.zeros((16,), jnp.float32) + jnp.sum(a1),
                           mask=lane0)
        plsc.store_scatter(s2b, [idx16],
                           jnp.zeros((16,), jnp.float32) + jnp.sum(a2),
                           mask=lane0)
        return 0

    lax.fori_loop(0, _RPW, row_body, 0)
    pltpu.sync_copy(s1b, s12_hbm.at[0, pl.ds(base, _RPW)])
    pltpu.sync_copy(s2b, s12_hbm.at[1, pl.ds(base, _RPW)])


@jax.jit
def kernel(output_stu, output_tch, label):
    B = output_stu.shape[0]
    pad = _CP - _C
    stu = jnp.pad(output_stu, ((0, 0), (0, pad)), constant_values=_NEG)
    tch = jnp.pad(output_tch, ((0, 0), (0, pad)), constant_values=_NEG)

    mesh = plsc.VectorSubcoreMesh(core_axis_name="c", subcore_axis_name="s")
    sc_call = functools.partial(
        pl.kernel, mesh=mesh,
        compiler_params=pltpu.CompilerParams(needs_layout_passes=False),
        out_type=jax.ShapeDtypeStruct((2, B), jnp.float32),
        scratch_types=[
            pltpu.VMEM((_CP,), jnp.float32),
            pltpu.VMEM((_CP,), jnp.float32),
            pltpu.VMEM((_RPW,), jnp.int32),
            pltpu.VMEM((_CP,), jnp.float32),
            pltpu.VMEM((_CP,), jnp.int32),
            pltpu.VMEM((_CP,), jnp.int32),
            pltpu.VMEM((_CP,), jnp.int32),
            pltpu.VMEM((_CP,), jnp.int32),
            pltpu.VMEM((_CP + 16,), jnp.int32),
            pltpu.VMEM((256,), jnp.int32),
            pltpu.VMEM((_RPW,), jnp.float32),
            pltpu.VMEM((_RPW,), jnp.float32),
        ],
    )(_sc_kernel)
    s12 = sc_call(tch, stu, label)
    s1 = s12[0]
    s2 = s12[1]

    stu3 = stu.reshape(B, 8, 128)
    tch3 = tch.reshape(B, 8, 128)
    grid_spec = pltpu.PrefetchScalarGridSpec(
        num_scalar_prefetch=1,
        grid=(B // _R,),
        in_specs=[
            pl.BlockSpec((_R, 8, 128), lambda i, lab: (i, 0, 0)),
            pl.BlockSpec((_R, 8, 128), lambda i, lab: (i, 0, 0)),
        ],
        out_specs=pl.BlockSpec((_R, 1, 4), lambda i, lab: (i, 0, 0)),
    )
    out = pl.pallas_call(
        _tc_kernel,
        grid_spec=grid_spec,
        out_shape=jax.ShapeDtypeStruct((B, 1, 4), jnp.float32),
    )(label, stu3, tch3)

    ce = out[:, 0, 0]
    partial = out[:, 0, 1]
    Z = out[:, 0, 2]
    m1lz = out[:, 0, 3]
    klrow = partial - (s1 / _T - m1lz * s2) / Z

    loss_stu = -jnp.mean(ce)
    loss_tch = jnp.sum(klrow) / (B * _C) * (_T * _T)
    return loss_stu * (1.0 - _ALPHA) + loss_tch * _ALPHA


# speedup vs baseline: 19.2571x; 1.1366x over previous
import functools
import jax
import jax.numpy as jnp
from jax import lax
from jax.experimental import pallas as pl
from jax.experimental.pallas import tpu as pltpu
from jax.experimental.pallas import tpu_sc as plsc

_ALPHA = 0.5
_T = 4.0
_C = 1000
_CP = 1024
_NEG = -1e30
_B = 4096
_NW = 32
_RPW = _B // _NW
_RB = 8

_R = 8


def _rsum(x):
    return jnp.sum(jnp.sum(x, axis=2, keepdims=True), axis=1, keepdims=True)


def _rmax(x):
    return jnp.max(jnp.max(x, axis=2, keepdims=True), axis=1, keepdims=True)


def _tc_kernel(label_ref, stu_ref, tch_ref, out_ref):
    step = pl.program_id(0)
    isub = lax.broadcasted_iota(jnp.int32, (_R, 8, 128), 1)
    ilane = lax.broadcasted_iota(jnp.int32, (_R, 8, 128), 2)
    iowr = lax.broadcasted_iota(jnp.int32, (_R, 1, 1), 0)
    lane4 = lax.broadcasted_iota(jnp.int32, (1, 1, 4), 2)

    labv = jnp.zeros((_R, 1, 1), jnp.int32)
    for r in range(_R):
        labv = jnp.where(iowr == r, label_ref[step * _R + r], labv)

    stu = stu_ref[...]
    tch = tch_ref[...]
    onehot = (isub == labv // 128) & (ilane == labv % 128)

    m1 = _rmax(stu)
    z1 = _rsum(jnp.exp(stu - m1))
    stu_lab = _rsum(jnp.where(onehot, stu, 0.0))
    ce = stu_lab - m1 - jnp.log(z1)

    zT = _rsum(jnp.exp((stu - m1) / _T))
    logzT = jnp.log(zT)
    L = (stu - m1) / _T - logzT

    m2 = _rmax(tch)
    E = jnp.exp((tch - m2) / _T)
    Z = _rsum(E)
    logp = (tch - m2) / _T - jnp.log(Z)
    plogp = _rsum(E * logp) / Z
    cross0 = _rsum(E * L) / Z
    E_lab = _rsum(jnp.where(onehot, E, 0.0))
    L_lab = _rsum(jnp.where(onehot, L, 0.0))

    partial = plogp - cross0 - (1.0 - E_lab) * L_lab / Z
    m1lz = m1 / _T + logzT

    out_ref[...] = jnp.where(lane4 == 0, ce,
                             jnp.where(lane4 == 1, partial,
                                       jnp.where(lane4 == 2, Z, m1lz)))


def _sc_kernel(tch_hbm, stu_hbm, lab_hbm, s12_hbm,
               tchb, stub, labb, Eb, ka, va, kb, vb, kshift, run, s1b, s2b):
    cid = lax.axis_index("c")
    sid = lax.axis_index("s")
    wid = sid * 2 + cid
    base = wid * _RPW
    pltpu.sync_copy(lab_hbm.at[pl.ds(base, _RPW)], labb)

    def row_body(bi, rr):
        i = _RB * bi + rr

        def mx(j, m):
            m0 = jnp.maximum(m[0], tchb[rr, pl.ds(64 * j, 16)])
            m1_ = jnp.maximum(m[1], tchb[rr, pl.ds(64 * j + 16, 16)])
            m2_ = jnp.maximum(m[2], tchb[rr, pl.ds(64 * j + 32, 16)])
            m3 = jnp.maximum(m[3], tchb[rr, pl.ds(64 * j + 48, 16)])
            return (m0, m1_, m2_, m3)
        mi = jnp.full((16,), _NEG, jnp.float32)
        mt = lax.fori_loop(0, 16, mx, (mi, mi, mi, mi))
        m2 = jnp.max(jnp.maximum(jnp.maximum(mt[0], mt[1]),
                                 jnp.maximum(mt[2], mt[3])))

        def build(j, _):
            for u in range(4):
                o = 64 * j + 16 * u
                t = tchb[rr, pl.ds(o, 16)]
                e = jnp.exp((t - m2) * (1.0 / _T))
                Eb[pl.ds(o, 16)] = e
                ka[pl.ds(o, 16)] = lax.shift_right_logical(
                    plsc.bitcast(e, jnp.int32), 16)
                va[pl.ds(o, 16)] = lax.iota(jnp.int32, 16) + o
            return 0
        lax.fori_loop(0, 16, build, 0)

        for ik, iv, ok, ov, sh, lastp in (
                (ka, va, kb, vb, 0, False),
                (kb, vb, ka, va, 8, True)):
            for q in range(16):
                run[pl.ds(16 * q, 16)] = jnp.zeros((16,), jnp.int32)

            def hist(j, _, ik=ik, sh=sh):
                for u in range(4):
                    o = 64 * j + 16 * u
                    d = lax.shift_right_logical(ik[pl.ds(o, 16)], sh) & 255
                    cnt, lastm = plsc.scan_count(d)
                    plsc.addupdate_scatter(run, [d], cnt, mask=lastm)
                return 0
            lax.fori_loop(0, 16, hist, 0)

            carry = jnp.int32(0)
            for q in range(16):
                h = run[pl.ds(16 * q, 16)]
                inc = plsc.cumsum(h)
                run[pl.ds(16 * q, 16)] = inc - h + carry
                carry = carry + jnp.sum(h)

            def place(j, _, ik=ik, iv=iv, ok=ok, ov=ov, sh=sh, lastp=lastp):
                for u in range(2):
                    o = 32 * j + 16 * u
                    k = ik[pl.ds(o, 16)]
                    v = iv[pl.ds(o, 16)]
                    d = lax.shift_right_logical(k, sh) & 255
                    cnt, lastm = plsc.scan_count(d)
                    dest = plsc.load_gather(run, [d]) + cnt - 1
                    plsc.store_scatter(ok, [dest], k)
                    plsc.store_scatter(ov, [dest], v)
                    if lastp:
                        plsc.store_scatter(kshift, [dest + 1], k)
                    plsc.addupdate_scatter(run, [d], cnt, mask=lastm)
                return 0
            lax.fori_loop(0, 32, place, 0)

        labv = plsc.load_gather(labb, [jnp.zeros((16,), jnp.int32) + i])
        labE = plsc.load_gather(Eb, [labv])
        labkey = lax.shift_right_logical(plsc.bitcast(labE, jnp.int32), 16)

        rridx = jnp.zeros((16,), jnp.int32) + rr

        def dsum(j, acc):
            a1, a2 = acc
            for u in range(4):
                o = 64 * j + 16 * u
                k = ka[pl.ds(o, 16)]
                kp = kshift[pl.ds(o, 16)]
                v = va[pl.ds(o, 16)]
                su = plsc.load_gather(stub, [rridx, v])
                e = plsc.bitcast(lax.shift_left(k, 16), jnp.float32)
                ep = plsc.bitcast(lax.shift_left(kp, 16), jnp.float32)
                de = jnp.where(k > labkey, ep - e, 0.0)
                a1 = a1 + de * su
                a2 = a2 + de
            return a1, a2
        a1, a2 = lax.fori_loop(
            0, 16, dsum,
            (jnp.zeros((16,), jnp.float32), jnp.zeros((16,), jnp.float32)))

        lane0 = lax.iota(jnp.int32, 16) == 0
        idx16 = jnp.zeros((16,), jnp.int32) + i
        plsc.store_scatter(s1b, [idx16],
                           jnp.zeros((16,), jnp.float32) + jnp.sum(a1),
                           mask=lane0)
        plsc.store_scatter(s2b, [idx16],
                           jnp.zeros((16,), jnp.float32) + jnp.sum(a2),
                           mask=lane0)

    def batch_body(bi, _):
        r0 = base + _RB * bi
        pltpu.sync_copy(tch_hbm.at[pl.ds(r0, _RB)], tchb)
        pltpu.sync_copy(stu_hbm.at[pl.ds(r0, _RB)], stub)

        def inner(rr, _):
            row_body(bi, rr)
            return 0
        lax.fori_loop(0, _RB, inner, 0)
        return 0

    lax.fori_loop(0, _RPW // _RB, batch_body, 0)
    pltpu.sync_copy(s1b, s12_hbm.at[0, pl.ds(base, _RPW)])
    pltpu.sync_copy(s2b, s12_hbm.at[1, pl.ds(base, _RPW)])


@jax.jit
def kernel(output_stu, output_tch, label):
    B = output_stu.shape[0]
    pad = _CP - _C
    stu = jnp.pad(output_stu, ((0, 0), (0, pad)), constant_values=_NEG)
    tch = jnp.pad(output_tch, ((0, 0), (0, pad)), constant_values=_NEG)

    mesh = plsc.VectorSubcoreMesh(core_axis_name="c", subcore_axis_name="s")
    sc_call = functools.partial(
        pl.kernel, mesh=mesh,
        compiler_params=pltpu.CompilerParams(needs_layout_passes=False),
        out_type=jax.ShapeDtypeStruct((2, B), jnp.float32),
        scratch_types=[
            pltpu.VMEM((_RB, _CP), jnp.float32),
            pltpu.VMEM((_RB, _CP), jnp.float32),
            pltpu.VMEM((_RPW,), jnp.int32),
            pltpu.VMEM((_CP,), jnp.float32),
            pltpu.VMEM((_CP,), jnp.int32),
            pltpu.VMEM((_CP,), jnp.int32),
            pltpu.VMEM((_CP,), jnp.int32),
            pltpu.VMEM((_CP,), jnp.int32),
            pltpu.VMEM((_CP + 16,), jnp.int32),
            pltpu.VMEM((256,), jnp.int32),
            pltpu.VMEM((_RPW,), jnp.float32),
            pltpu.VMEM((_RPW,), jnp.float32),
        ],
    )(_sc_kernel)
    s12 = sc_call(tch, stu, label)
    s1 = s12[0]
    s2 = s12[1]

    stu3 = stu.reshape(B, 8, 128)
    tch3 = tch.reshape(B, 8, 128)
    grid_spec = pltpu.PrefetchScalarGridSpec(
        num_scalar_prefetch=1,
        grid=(B // _R,),
        in_specs=[
            pl.BlockSpec((_R, 8, 128), lambda i, lab: (i, 0, 0)),
            pl.BlockSpec((_R, 8, 128), lambda i, lab: (i, 0, 0)),
        ],
        out_specs=pl.BlockSpec((_R, 1, 4), lambda i, lab: (i, 0, 0)),
    )
    out = pl.pallas_call(
        _tc_kernel,
        grid_spec=grid_spec,
        out_shape=jax.ShapeDtypeStruct((B, 1, 4), jnp.float32),
    )(label, stu3, tch3)

    ce = out[:, 0, 0]
    partial = out[:, 0, 1]
    Z = out[:, 0, 2]
    m1lz = out[:, 0, 3]
    klrow = partial - (s1 / _T - m1lz * s2) / Z

    loss_stu = -jnp.mean(ce)
    loss_tch = jnp.sum(klrow) / (B * _C) * (_T * _T)
    return loss_stu * (1.0 - _ALPHA) + loss_tch * _ALPHA


# speedup vs baseline: 19.3141x; 1.0030x over previous
import functools
import jax
import jax.numpy as jnp
from jax import lax
from jax.experimental import pallas as pl
from jax.experimental.pallas import tpu as pltpu
from jax.experimental.pallas import tpu_sc as plsc

_ALPHA = 0.5
_T = 4.0
_C = 1000
_CP = 1024
_NEG = -1e30
_B = 4096
_NW = 32
_RPW = _B // _NW
_RB = 8

_R = 8


def _rsum(x):
    return jnp.sum(jnp.sum(x, axis=2, keepdims=True), axis=1, keepdims=True)


def _rmax(x):
    return jnp.max(jnp.max(x, axis=2, keepdims=True), axis=1, keepdims=True)


def _tc_kernel(label_ref, stu_ref, tch_ref, out_ref):
    step = pl.program_id(0)
    isub = lax.broadcasted_iota(jnp.int32, (_R, 8, 128), 1)
    ilane = lax.broadcasted_iota(jnp.int32, (_R, 8, 128), 2)
    iowr = lax.broadcasted_iota(jnp.int32, (_R, 1, 1), 0)
    lane4 = lax.broadcasted_iota(jnp.int32, (1, 1, 4), 2)

    labv = jnp.zeros((_R, 1, 1), jnp.int32)
    for r in range(_R):
        labv = jnp.where(iowr == r, label_ref[step * _R + r], labv)

    stu = stu_ref[...]
    tch = tch_ref[...]
    onehot = (isub == labv // 128) & (ilane == labv % 128)

    m1 = _rmax(stu)
    z1 = _rsum(jnp.exp(stu - m1))
    stu_lab = _rsum(jnp.where(onehot, stu, 0.0))
    ce = stu_lab - m1 - jnp.log(z1)

    zT = _rsum(jnp.exp((stu - m1) / _T))
    logzT = jnp.log(zT)
    L = (stu - m1) / _T - logzT

    m2 = _rmax(tch)
    E = jnp.exp((tch - m2) / _T)
    Z = _rsum(E)
    logp = (tch - m2) / _T - jnp.log(Z)
    plogp = _rsum(E * logp) / Z
    cross0 = _rsum(E * L) / Z
    E_lab = _rsum(jnp.where(onehot, E, 0.0))
    L_lab = _rsum(jnp.where(onehot, L, 0.0))

    partial = plogp - cross0 - (1.0 - E_lab) * L_lab / Z
    m1lz = m1 / _T + logzT

    out_ref[...] = jnp.where(lane4 == 0, ce,
                             jnp.where(lane4 == 1, partial,
                                       jnp.where(lane4 == 2, Z, m1lz)))


def _sc_kernel(tch_hbm, stu_hbm, lab_hbm, s12_hbm,
               tchb, stub, labb, Eb, ka, va, kb, vb, kshift, run, s1b, s2b):
    cid = lax.axis_index("c")
    sid = lax.axis_index("s")
    wid = sid * 2 + cid
    base = wid * _RPW
    pltpu.sync_copy(lab_hbm.at[pl.ds(base, _RPW)], labb)

    def row_body(bi, rr):
        i = _RB * bi + rr

        def mx(j, m):
            m0 = jnp.maximum(m[0], tchb[rr, pl.ds(64 * j, 16)])
            m1_ = jnp.maximum(m[1], tchb[rr, pl.ds(64 * j + 16, 16)])
            m2_ = jnp.maximum(m[2], tchb[rr, pl.ds(64 * j + 32, 16)])
            m3 = jnp.maximum(m[3], tchb[rr, pl.ds(64 * j + 48, 16)])
            return (m0, m1_, m2_, m3)
        mi = jnp.full((16,), _NEG, jnp.float32)
        mt = lax.fori_loop(0, 16, mx, (mi, mi, mi, mi))
        m2 = jnp.max(jnp.maximum(jnp.maximum(mt[0], mt[1]),
                                 jnp.maximum(mt[2], mt[3])))

        def build(j, _):
            for u in range(4):
                o = 64 * j + 16 * u
                t = tchb[rr, pl.ds(o, 16)]
                e = jnp.exp((t - m2) * (1.0 / _T))
                Eb[pl.ds(o, 16)] = e
                ka[pl.ds(o, 16)] = lax.shift_right_logical(
                    plsc.bitcast(e, jnp.int32), 16)
                va[pl.ds(o, 16)] = lax.iota(jnp.int32, 16) + o
            return 0
        lax.fori_loop(0, 16, build, 0)

        for ik, iv, ok, ov, sh, lastp in (
                (ka, va, kb, vb, 0, False),
                (kb, vb, ka, va, 8, True)):
            for q in range(16):
                run[pl.ds(16 * q, 16)] = jnp.zeros((16,), jnp.int32)

            def hist(j, _, ik=ik, sh=sh):
                for u in range(4):
                    o = 64 * j + 16 * u
                    d = lax.shift_right_logical(ik[pl.ds(o, 16)], sh) & 255
                    cnt, lastm = plsc.scan_count(d)
                    plsc.addupdate_scatter(run, [d], cnt, mask=lastm)
                return 0
            lax.fori_loop(0, 16, hist, 0)

            carry = jnp.int32(0)
            for q in range(16):
                h = run[pl.ds(16 * q, 16)]
                inc = plsc.cumsum(h)
                run[pl.ds(16 * q, 16)] = inc - h + carry
                carry = carry + jnp.sum(h)

            def place(j, _, ik=ik, iv=iv, ok=ok, ov=ov, sh=sh, lastp=lastp):
                for u in range(4):
                    o = 64 * j + 16 * u
                    k = ik[pl.ds(o, 16)]
                    v = iv[pl.ds(o, 16)]
                    d = lax.shift_right_logical(k, sh) & 255
                    cnt, lastm = plsc.scan_count(d)
                    dest = plsc.load_gather(run, [d]) + cnt - 1
                    plsc.store_scatter(ok, [dest], k)
                    plsc.store_scatter(ov, [dest], v)
                    if lastp:
                        plsc.store_scatter(kshift, [dest + 1], k)
                    plsc.addupdate_scatter(run, [d], cnt, mask=lastm)
                return 0
            lax.fori_loop(0, 16, place, 0)

        labv = plsc.load_gather(labb, [jnp.zeros((16,), jnp.int32) + i])
        labE = plsc.load_gather(Eb, [labv])
        labkey = lax.shift_right_logical(plsc.bitcast(labE, jnp.int32), 16)

        rridx = jnp.zeros((16,), jnp.int32) + rr

        def dsum(j, acc):
            a1, a2 = acc
            for u in range(4):
                o = 64 * j + 16 * u
                k = ka[pl.ds(o, 16)]
                kp = kshift[pl.ds(o, 16)]
                v = va[pl.ds(o, 16)]
                su = plsc.load_gather(stub, [rridx, v])
                e = plsc.bitcast(lax.shift_left(k, 16), jnp.float32)
                ep = plsc.bitcast(lax.shift_left(kp, 16), jnp.float32)
                de = jnp.where(k > labkey, ep - e, 0.0)
                a1 = a1 + de * su
                a2 = a2 + de
            return a1, a2
        a1, a2 = lax.fori_loop(
            0, 16, dsum,
            (jnp.zeros((16,), jnp.float32), jnp.zeros((16,), jnp.float32)))

        lane0 = lax.iota(jnp.int32, 16) == 0
        idx16 = jnp.zeros((16,), jnp.int32) + i
        plsc.store_scatter(s1b, [idx16],
                           jnp.zeros((16,), jnp.float32) + jnp.sum(a1),
                           mask=lane0)
        plsc.store_scatter(s2b, [idx16],
                           jnp.zeros((16,), jnp.float32) + jnp.sum(a2),
                           mask=lane0)

    def batch_body(bi, _):
        r0 = base + _RB * bi
        pltpu.sync_copy(tch_hbm.at[pl.ds(r0, _RB)], tchb)
        pltpu.sync_copy(stu_hbm.at[pl.ds(r0, _RB)], stub)

        def inner(rr, _):
            row_body(bi, rr)
            return 0
        lax.fori_loop(0, _RB, inner, 0)
        return 0

    lax.fori_loop(0, _RPW // _RB, batch_body, 0)
    pltpu.sync_copy(s1b, s12_hbm.at[0, pl.ds(base, _RPW)])
    pltpu.sync_copy(s2b, s12_hbm.at[1, pl.ds(base, _RPW)])


@jax.jit
def kernel(output_stu, output_tch, label):
    B = output_stu.shape[0]
    pad = _CP - _C
    stu = jnp.pad(output_stu, ((0, 0), (0, pad)), constant_values=_NEG)
    tch = jnp.pad(output_tch, ((0, 0), (0, pad)), constant_values=_NEG)

    mesh = plsc.VectorSubcoreMesh(core_axis_name="c", subcore_axis_name="s")
    sc_call = functools.partial(
        pl.kernel, mesh=mesh,
        compiler_params=pltpu.CompilerParams(needs_layout_passes=False),
        out_type=jax.ShapeDtypeStruct((2, B), jnp.float32),
        scratch_types=[
            pltpu.VMEM((_RB, _CP), jnp.float32),
            pltpu.VMEM((_RB, _CP), jnp.float32),
            pltpu.VMEM((_RPW,), jnp.int32),
            pltpu.VMEM((_CP,), jnp.float32),
            pltpu.VMEM((_CP,), jnp.int32),
            pltpu.VMEM((_CP,), jnp.int32),
            pltpu.VMEM((_CP,), jnp.int32),
            pltpu.VMEM((_CP,), jnp.int32),
            pltpu.VMEM((_CP + 16,), jnp.int32),
            pltpu.VMEM((256,), jnp.int32),
            pltpu.VMEM((_RPW,), jnp.float32),
            pltpu.VMEM((_RPW,), jnp.float32),
        ],
    )(_sc_kernel)
    s12 = sc_call(tch, stu, label)
    s1 = s12[0]
    s2 = s12[1]

    stu3 = stu.reshape(B, 8, 128)
    tch3 = tch.reshape(B, 8, 128)
    grid_spec = pltpu.PrefetchScalarGridSpec(
        num_scalar_prefetch=1,
        grid=(B // _R,),
        in_specs=[
            pl.BlockSpec((_R, 8, 128), lambda i, lab: (i, 0, 0)),
            pl.BlockSpec((_R, 8, 128), lambda i, lab: (i, 0, 0)),
        ],
        out_specs=pl.BlockSpec((_R, 1, 4), lambda i, lab: (i, 0, 0)),
    )
    out = pl.pallas_call(
        _tc_kernel,
        grid_spec=grid_spec,
        out_shape=jax.ShapeDtypeStruct((B, 1, 4), jnp.float32),
    )(label, stu3, tch3)

    ce = out[:, 0, 0]
    partial = out[:, 0, 1]
    Z = out[:, 0, 2]
    m1lz = out[:, 0, 3]
    klrow = partial - (s1 / _T - m1lz * s2) / Z

    loss_stu = -jnp.mean(ce)
    loss_tch = jnp.sum(klrow) / (B * _C) * (_T * _T)
    return loss_stu * (1.0 - _ALPHA) + loss_tch * _ALPHA
